# baseline (speedup 1.0000x reference)
import functools

import jax
import jax.numpy as jnp
from jax import lax
from jax.experimental import pallas as pl
from jax.experimental.pallas import tpu as pltpu
from jax.experimental.pallas import tpu_sc as plsc


def _make_sc_kernel(V1, V0, M1, M0, E):
    info = plsc.get_sparse_core_info()
    NC, NS = info.num_cores, info.num_subcores
    NW = NC * NS
    M = M1 + M0

    CH = 128
    NB = 82
    assert E % CH == 0
    NCHUNK = E // CH
    per_w = -(-NCHUNK // NW)
    NWAVES = -(-per_w // NB)
    if NWAVES % 2:
        NWAVES += 1
    per_w_eff = NWAVES * NB
    assert per_w_eff <= NCHUNK
    WELEM = NB * CH

    S = (M1 // (NS * 256)) * 256
    assert M1 == M0, "equal-sized regions expected"
    T1 = M1 - NS * S
    T0 = M0 - NS * S
    BSZ = S // 4
    NG = BSZ // CH
    assert S % 256 == 0 and T1 % 8 == 0 and T0 % 8 == 0 and M1 % 8 == 0
    assert max(T1, T0) <= BSZ and BSZ % 16 == 0

    mesh = plsc.VectorSubcoreMesh(core_axis_name="c", subcore_axis_name="s")

    @functools.partial(
        pl.kernel,
        mesh=mesh,
        out_type=(jax.ShapeDtypeStruct((E,), jnp.float32),
                  jax.ShapeDtypeStruct((M,), jnp.float32),
                  jax.ShapeDtypeStruct((16,), jnp.int32)),
        scratch_types=[
            pltpu.VMEM_SHARED((M,), jnp.float32),
            pltpu.VMEM((BSZ,), jnp.int32),
            pltpu.VMEM((BSZ,), jnp.float32),
            pltpu.VMEM((2 * WELEM,), jnp.int32),
            pltpu.VMEM((2 * WELEM,), jnp.float32),
            pltpu.VMEM((16,), jnp.int32),
            pltpu.SemaphoreType.DMA,
            pltpu.SemaphoreType.DMA,
            pltpu.SemaphoreType.DMA,
            pltpu.SemaphoreType.DMA,
            pltpu.SemaphoreType.DMA,
            pltpu.SemaphoreType.DMA,
        ],
    )
    def run(vals1_hbm, vals0_hbm, ord1_hbm, ord0_hbm, c_hbm,
            out_hbm, bounce_hbm, flags_hbm,
            lin_sp, ord_v, rows_v, c_v, out_v, flag_v,
            sem_st, sem_c, sem_g0, sem_g1, sem_o, sem_b):
        t = lax.axis_index("s")
        cid = lax.axis_index("c")
        wid = t * NC + cid

        w_base = jnp.minimum(wid * per_w, NCHUNK - per_w_eff)

        def elem_base(v):
            return pl.multiple_of((w_base + v * NB) * CH, CH)

        pltpu.async_copy(c_hbm.at[pl.ds(elem_base(0), WELEM)],
                         c_v.at[pl.ds(0, WELEM)], sem_c)

        pltpu.sync_copy(ord1_hbm.at[pl.ds(0, 8)], flag_v.at[pl.ds(0, 8)])
        magic = (flag_v[pl.ds(0, 16)][0] * jnp.int32(-1640531527)
                 + jnp.int32(0x5AC3B1F))

        def gather_block(tbl, n):
            nfull = n // CH
            rem = n - nfull * CH

            def fire(g, _):
                o = pl.multiple_of(g * CH, CH)
                pltpu.async_copy(tbl.at[ord_v.at[pl.ds(o, CH)]],
                                 rows_v.at[pl.ds(o, CH)], sem_st)
                return 0
            lax.fori_loop(0, nfull, fire, 0)
            if rem:
                pltpu.async_copy(
                    tbl.at[ord_v.at[pl.ds(nfull * CH, rem)]],
                    rows_v.at[pl.ds(nfull * CH, rem)], sem_st)

            def drain(g, _):
                pltpu.make_async_copy(tbl.at[ord_v.at[pl.ds(0, CH)]],
                                      rows_v.at[pl.ds(0, CH)], sem_st).wait()
                return 0
            lax.fori_loop(0, nfull, drain, 0)
            if rem:
                pltpu.make_async_copy(
                    tbl.at[ord_v.at[pl.ds(0, rem)]],
                    rows_v.at[pl.ds(0, rem)], sem_st).wait()

        def stage_region(src_ref, tbl, base, T):
            bounce_pend = []
            for half in range(4):
                start = t * S + half * BSZ
                pltpu.sync_copy(src_ref.at[pl.ds(start, BSZ)],
                                ord_v.at[pl.ds(0, BSZ)])
                gather_block(tbl, BSZ)
                if bounce_pend:
                    bounce_pend.pop().wait()
                bounce_pend.append(
                    pltpu.async_copy(rows_v.at[pl.ds(0, BSZ)],
                                     bounce_hbm.at[pl.ds(base + start, BSZ)],
                                     sem_b))
                pltpu.sync_copy(rows_v.at[pl.ds(0, BSZ)],
                                lin_sp.at[pl.ds(base + start, BSZ)])
            if T:
                @pl.when(t == 0)
                def _():
                    tb = NS * S
                    pltpu.sync_copy(src_ref.at[pl.ds(tb, T)],
                                    ord_v.at[pl.ds(0, T)])
                    gather_block(tbl, T)
                    pltpu.sync_copy(rows_v.at[pl.ds(0, T)],
                                    bounce_hbm.at[pl.ds(base + tb, T)])
                    pltpu.sync_copy(rows_v.at[pl.ds(0, T)],
                                    lin_sp.at[pl.ds(base + tb, T)])
            bounce_pend.pop().wait()

        @pl.when(cid == 0)
        def _():
            stage_region(ord1_hbm, vals1_hbm, 0, T1)

        @pl.when(cid == 1)
        def _():
            stage_region(ord0_hbm, vals0_hbm, M1, T0)

        plsc.subcore_barrier()

        flag_v[pl.ds(0, 16)] = jnp.full((16,), magic, jnp.int32)
        pltpu.sync_copy(flag_v.at[pl.ds(0, 8)],
                        flags_hbm.at[pl.ds(pl.multiple_of(cid * 8, 8), 8)])

        def poll_step(i, done):
            @pl.when(done == 0)
            def _():
                pltpu.sync_copy(
                    flags_hbm.at[pl.ds(pl.multiple_of((1 - cid) * 8, 8), 8)],
                    flag_v.at[pl.ds(8, 8)])
            v = flag_v[pl.ds(0, 16)][8]
            return jnp.where(v == magic, jnp.int32(1), done)

        lax.fori_loop(0, 384, poll_step, jnp.int32(0))

        other = pl.multiple_of((1 - cid) * M1, 8)
        for half in range(4):
            start = t * S + half * BSZ
            pltpu.sync_copy(bounce_hbm.at[pl.ds(other + start, BSZ)],
                            rows_v.at[pl.ds(0, BSZ)])
            pltpu.sync_copy(rows_v.at[pl.ds(0, BSZ)],
                            lin_sp.at[pl.ds(other + start, BSZ)])
        if T1 or T0:
            @pl.when(t == 0)
            def _():
                tb = NS * S
                TT = max(T1, T0)
                pltpu.sync_copy(bounce_hbm.at[pl.ds(other + tb, TT)],
                                rows_v.at[pl.ds(0, TT)])
                pltpu.sync_copy(rows_v.at[pl.ds(0, TT)],
                                lin_sp.at[pl.ds(other + tb, TT)])

        plsc.subcore_barrier()

        def issue_gathers(v, buf):
            co = buf * WELEM
            sem = sem_g1 if buf else sem_g0
            for b in range(NB):
                pltpu.async_copy(
                    lin_sp.at[c_v.at[pl.ds(co + b * CH, CH)]],
                    out_v.at[pl.ds(co + b * CH, CH)], sem)

        def drain_gathers(buf):
            co = buf * WELEM
            sem = sem_g1 if buf else sem_g0
            for b in range(NB):
                pltpu.make_async_copy(
                    lin_sp.at[c_v.at[pl.ds(co + b * CH, CH)]],
                    out_v.at[pl.ds(co + b * CH, CH)], sem).wait()

        def wait_c(v, buf):
            pltpu.make_async_copy(c_hbm.at[pl.ds(elem_base(v), WELEM)],
                                  c_v.at[pl.ds(buf * WELEM, WELEM)],
                                  sem_c).wait()

        def load_c(v, buf):
            pltpu.async_copy(c_hbm.at[pl.ds(elem_base(v), WELEM)],
                             c_v.at[pl.ds(buf * WELEM, WELEM)], sem_c)

        def store_out(v, buf):
            pltpu.async_copy(out_v.at[pl.ds(buf * WELEM, WELEM)],
                             out_hbm.at[pl.ds(elem_base(v), WELEM)], sem_o)

        def wait_store(v, buf):
            pltpu.make_async_copy(out_v.at[pl.ds(buf * WELEM, WELEM)],
                                  out_hbm.at[pl.ds(elem_base(v), WELEM)],
                                  sem_o).wait()

        def pair_body(u, _):
            v0 = u * 2
            v1 = v0 + 1
            wait_c(v0, 0)

            @pl.when(u >= 1)
            def _():
                wait_store(v0 - 2, 0)
            issue_gathers(v0, 0)

            @pl.when(u >= 1)
            def _():
                drain_gathers(1)
                store_out(v0 - 1, 1)

            @pl.when(v1 < NWAVES)
            def _():
                load_c(v1, 1)
            wait_c(v1, 1)

            @pl.when(u >= 1)
            def _():
                wait_store(v1 - 2, 1)
            issue_gathers(v1, 1)
            drain_gathers(0)
            store_out(v0, 0)

            @pl.when(v1 + 1 < NWAVES)
            def _():
                load_c(v1 + 1, 0)
            return 0

        lax.fori_loop(0, NWAVES // 2, pair_body, 0)

        vl = NWAVES - 1
        drain_gathers(1)
        wait_store(vl - 1, 0)
        store_out(vl, 1)
        wait_store(vl, 1)

    return run


@jax.jit
def kernel(layer0_values, layer1_values, per_layer_ordinals0,
           per_layer_ordinals1, concatenated_ordinals):
    V0 = layer0_values.shape[0]
    V1 = layer1_values.shape[0]
    M0 = per_layer_ordinals0.shape[0]
    M1 = per_layer_ordinals1.shape[0]
    E = concatenated_ordinals.shape[0]
    run = _make_sc_kernel(V1, V0, M1, M0, E)
    out, _bounce, _flags = run(layer1_values, layer0_values,
                               per_layer_ordinals1, per_layer_ordinals0,
                               concatenated_ordinals)
    return out


# speedup vs baseline: 1.0447x; 1.0447x over previous
import functools

import jax
import jax.numpy as jnp
from jax import lax
from jax.experimental import pallas as pl
from jax.experimental.pallas import tpu as pltpu
from jax.experimental.pallas import tpu_sc as plsc


def _make_sc_kernel(V1, V0, M1, M0, E):
    info = plsc.get_sparse_core_info()
    NC, NS = info.num_cores, info.num_subcores
    NW = NC * NS
    M = M1 + M0

    CH = 128
    NB = 70
    assert E % CH == 0
    NCHUNK = E // CH
    per_w = -(-NCHUNK // NW)
    NWAVES = -(-per_w // NB)
    if NWAVES % 2:
        NWAVES += 1
    per_w_eff = NWAVES * NB
    assert per_w_eff <= NCHUNK
    WELEM = NB * CH

    S = (M1 // (NS * 256)) * 256
    assert M1 == M0, "equal-sized regions expected"
    T1 = M1 - NS * S
    T0 = M0 - NS * S
    BSZ = S // 2
    NG = BSZ // CH
    assert S % 256 == 0 and T1 % 8 == 0 and T0 % 8 == 0 and M1 % 8 == 0
    assert max(T1, T0) <= BSZ and BSZ % 16 == 0

    mesh = plsc.VectorSubcoreMesh(core_axis_name="c", subcore_axis_name="s")

    @functools.partial(
        pl.kernel,
        mesh=mesh,
        out_type=(jax.ShapeDtypeStruct((E,), jnp.float32),
                  jax.ShapeDtypeStruct((M,), jnp.float32),
                  jax.ShapeDtypeStruct((16,), jnp.int32)),
        scratch_types=[
            pltpu.VMEM_SHARED((M,), jnp.float32),
            pltpu.VMEM((BSZ,), jnp.int32),
            pltpu.VMEM((BSZ,), jnp.float32),
            pltpu.VMEM((2 * WELEM,), jnp.int32),
            pltpu.VMEM((2 * WELEM,), jnp.float32),
            pltpu.VMEM((16,), jnp.int32),
            pltpu.SemaphoreType.DMA,
            pltpu.SemaphoreType.DMA,
            pltpu.SemaphoreType.DMA,
            pltpu.SemaphoreType.DMA,
            pltpu.SemaphoreType.DMA,
            pltpu.SemaphoreType.DMA,
        ],
    )
    def run(vals1_hbm, vals0_hbm, ord1_hbm, ord0_hbm, c_hbm,
            out_hbm, bounce_hbm, flags_hbm,
            lin_sp, ord_v, rows_v, c_v, out_v, flag_v,
            sem_st, sem_c, sem_g0, sem_g1, sem_o, sem_b):
        t = lax.axis_index("s")
        cid = lax.axis_index("c")
        wid = t * NC + cid

        w_base = jnp.minimum(wid * per_w, NCHUNK - per_w_eff)

        def elem_base(v):
            return pl.multiple_of((w_base + v * NB) * CH, CH)

        pltpu.async_copy(c_hbm.at[pl.ds(elem_base(0), WELEM)],
                         c_v.at[pl.ds(0, WELEM)], sem_c)

        pltpu.sync_copy(ord1_hbm.at[pl.ds(0, 8)], flag_v.at[pl.ds(0, 8)])
        magic = (flag_v[pl.ds(0, 16)][0] * jnp.int32(-1640531527)
                 + jnp.int32(0x5AC3B1F))

        def gather_block(tbl, n):
            nfull = n // CH
            rem = n - nfull * CH

            def fire(g, _):
                o = pl.multiple_of(g * CH, CH)
                pltpu.async_copy(tbl.at[ord_v.at[pl.ds(o, CH)]],
                                 rows_v.at[pl.ds(o, CH)], sem_st)
                return 0
            lax.fori_loop(0, nfull, fire, 0)
            if rem:
                pltpu.async_copy(
                    tbl.at[ord_v.at[pl.ds(nfull * CH, rem)]],
                    rows_v.at[pl.ds(nfull * CH, rem)], sem_st)

            def drain(g, _):
                pltpu.make_async_copy(tbl.at[ord_v.at[pl.ds(0, CH)]],
                                      rows_v.at[pl.ds(0, CH)], sem_st).wait()
                return 0
            lax.fori_loop(0, nfull, drain, 0)
            if rem:
                pltpu.make_async_copy(
                    tbl.at[ord_v.at[pl.ds(0, rem)]],
                    rows_v.at[pl.ds(0, rem)], sem_st).wait()

        def stage_region(src_ref, tbl, base, T):
            bounce_pend = []
            for half in range(2):
                start = t * S + half * BSZ
                pltpu.sync_copy(src_ref.at[pl.ds(start, BSZ)],
                                ord_v.at[pl.ds(0, BSZ)])
                gather_block(tbl, BSZ)
                if bounce_pend:
                    bounce_pend.pop().wait()
                bounce_pend.append(
                    pltpu.async_copy(rows_v.at[pl.ds(0, BSZ)],
                                     bounce_hbm.at[pl.ds(base + start, BSZ)],
                                     sem_b))
                pltpu.sync_copy(rows_v.at[pl.ds(0, BSZ)],
                                lin_sp.at[pl.ds(base + start, BSZ)])
            if T:
                @pl.when(t == 0)
                def _():
                    tb = NS * S
                    pltpu.sync_copy(src_ref.at[pl.ds(tb, T)],
                                    ord_v.at[pl.ds(0, T)])
                    gather_block(tbl, T)
                    pltpu.sync_copy(rows_v.at[pl.ds(0, T)],
                                    bounce_hbm.at[pl.ds(base + tb, T)])
                    pltpu.sync_copy(rows_v.at[pl.ds(0, T)],
                                    lin_sp.at[pl.ds(base + tb, T)])
            bounce_pend.pop().wait()

        @pl.when(cid == 0)
        def _():
            stage_region(ord1_hbm, vals1_hbm, 0, T1)

        @pl.when(cid == 1)
        def _():
            stage_region(ord0_hbm, vals0_hbm, M1, T0)

        plsc.subcore_barrier()

        flag_v[pl.ds(0, 16)] = jnp.full((16,), magic, jnp.int32)
        pltpu.sync_copy(flag_v.at[pl.ds(0, 8)],
                        flags_hbm.at[pl.ds(pl.multiple_of(cid * 8, 8), 8)])

        def poll_step(i, done):
            @pl.when(done == 0)
            def _():
                pltpu.sync_copy(
                    flags_hbm.at[pl.ds(pl.multiple_of((1 - cid) * 8, 8), 8)],
                    flag_v.at[pl.ds(8, 8)])
            v = flag_v[pl.ds(0, 16)][8]
            return jnp.where(v == magic, jnp.int32(1), done)

        lax.fori_loop(0, 384, poll_step, jnp.int32(0))

        other = pl.multiple_of((1 - cid) * M1, 8)
        for half in range(2):
            start = t * S + half * BSZ
            pltpu.sync_copy(bounce_hbm.at[pl.ds(other + start, BSZ)],
                            rows_v.at[pl.ds(0, BSZ)])
            pltpu.sync_copy(rows_v.at[pl.ds(0, BSZ)],
                            lin_sp.at[pl.ds(other + start, BSZ)])
        if T1 or T0:
            @pl.when(t == 0)
            def _():
                tb = NS * S
                TT = max(T1, T0)
                pltpu.sync_copy(bounce_hbm.at[pl.ds(other + tb, TT)],
                                rows_v.at[pl.ds(0, TT)])
                pltpu.sync_copy(rows_v.at[pl.ds(0, TT)],
                                lin_sp.at[pl.ds(other + tb, TT)])

        plsc.subcore_barrier()

        def issue_gathers(v, buf):
            co = buf * WELEM
            sem = sem_g1 if buf else sem_g0
            for b in range(NB):
                pltpu.async_copy(
                    lin_sp.at[c_v.at[pl.ds(co + b * CH, CH)]],
                    out_v.at[pl.ds(co + b * CH, CH)], sem)

        def drain_gathers(buf):
            co = buf * WELEM
            sem = sem_g1 if buf else sem_g0
            for b in range(NB):
                pltpu.make_async_copy(
                    lin_sp.at[c_v.at[pl.ds(co + b * CH, CH)]],
                    out_v.at[pl.ds(co + b * CH, CH)], sem).wait()

        def wait_c(v, buf):
            pltpu.make_async_copy(c_hbm.at[pl.ds(elem_base(v), WELEM)],
                                  c_v.at[pl.ds(buf * WELEM, WELEM)],
                                  sem_c).wait()

        def load_c(v, buf):
            pltpu.async_copy(c_hbm.at[pl.ds(elem_base(v), WELEM)],
                             c_v.at[pl.ds(buf * WELEM, WELEM)], sem_c)

        def store_out(v, buf):
            pltpu.async_copy(out_v.at[pl.ds(buf * WELEM, WELEM)],
                             out_hbm.at[pl.ds(elem_base(v), WELEM)], sem_o)

        def wait_store(v, buf):
            pltpu.make_async_copy(out_v.at[pl.ds(buf * WELEM, WELEM)],
                                  out_hbm.at[pl.ds(elem_base(v), WELEM)],
                                  sem_o).wait()

        def pair_body(u, _):
            v0 = u * 2
            v1 = v0 + 1
            wait_c(v0, 0)

            @pl.when(u >= 1)
            def _():
                wait_store(v0 - 2, 0)
            issue_gathers(v0, 0)

            @pl.when(u >= 1)
            def _():
                drain_gathers(1)
                store_out(v0 - 1, 1)

            @pl.when(v1 < NWAVES)
            def _():
                load_c(v1, 1)
            wait_c(v1, 1)

            @pl.when(u >= 1)
            def _():
                wait_store(v1 - 2, 1)
            issue_gathers(v1, 1)
            drain_gathers(0)
            store_out(v0, 0)

            @pl.when(v1 + 1 < NWAVES)
            def _():
                load_c(v1 + 1, 0)
            return 0

        lax.fori_loop(0, NWAVES // 2, pair_body, 0)

        vl = NWAVES - 1
        drain_gathers(1)
        wait_store(vl - 1, 0)
        store_out(vl, 1)
        wait_store(vl, 1)

    return run


@jax.jit
def kernel(layer0_values, layer1_values, per_layer_ordinals0,
           per_layer_ordinals1, concatenated_ordinals):
    V0 = layer0_values.shape[0]
    V1 = layer1_values.shape[0]
    M0 = per_layer_ordinals0.shape[0]
    M1 = per_layer_ordinals1.shape[0]
    E = concatenated_ordinals.shape[0]
    run = _make_sc_kernel(V1, V0, M1, M0, E)
    out, _bounce, _flags = run(layer1_values, layer0_values,
                               per_layer_ordinals1, per_layer_ordinals0,
                               concatenated_ordinals)
    return out
